# speedup vs baseline: 7.1799x; 7.1799x over previous
import functools

import jax
import jax.numpy as jnp
from jax import lax
from jax.experimental import pallas as pl
from jax.experimental.pallas import tpu as pltpu
from jax.experimental.pallas import tpu_sc as plsc

N = 10000
D_IN = 256
D_OUT = 256
E = 160000

NP = 10240
ROWS = NP // 16
EP = 163840
CHUNK = 128
EPT_AGG = EP // 16
EPT_DEG = EP // 32
BLK = 512
RB = NP // BLK

_MESH = plsc.VectorSubcoreMesh(core_axis_name="c", subcore_axis_name="s")


@functools.partial(
    pl.kernel,
    out_type=jax.ShapeDtypeStruct((2 * NP,), jnp.float32),
    mesh=_MESH,
    scratch_types=[
        pltpu.VMEM((CHUNK,), jnp.int32),
        pltpu.VMEM((CHUNK,), jnp.float32),
        pltpu.VMEM_SHARED((NP,), jnp.float32),
    ],
)
def _deg_kernel(dst_hbm, z1_hbm, ones_hbm, deg_hbm, idx_v, ones_v, acc_sh):
    c = lax.axis_index("c")
    s = lax.axis_index("s")
    pltpu.sync_copy(z1_hbm.at[pl.ds(s * ROWS, ROWS)],
                    acc_sh.at[pl.ds(s * ROWS, ROWS)])
    pltpu.sync_copy(ones_hbm, ones_v)
    plsc.subcore_barrier()

    base = (c * 16 + s) * EPT_DEG

    def body(j, carry):
        off = base + j * CHUNK
        pltpu.sync_copy(dst_hbm.at[pl.ds(off, CHUNK)], idx_v)
        pltpu.sync_copy(ones_v, acc_sh.at[idx_v], add=True)
        return carry

    lax.fori_loop(0, EPT_DEG // CHUNK, body, 0)
    plsc.subcore_barrier()
    pltpu.sync_copy(acc_sh.at[pl.ds(s * ROWS, ROWS)],
                    deg_hbm.at[pl.ds(c * NP + s * ROWS, ROWS)])


@functools.partial(
    pl.kernel,
    out_type=jax.ShapeDtypeStruct((2 * NP, 128), jnp.float32),
    mesh=_MESH,
    scratch_types=[
        pltpu.VMEM((CHUNK,), jnp.int32),
        pltpu.VMEM((CHUNK,), jnp.int32),
        pltpu.VMEM((CHUNK, 128), jnp.float32),
        pltpu.VMEM_SHARED((NP, 128), jnp.float32),
        pltpu.SemaphoreType.DMA,
    ],
)
def _agg_kernel(table_hbm, src2_hbm, dst_hbm, z2_hbm, acc_hbm,
                sidx_v, didx_v, rows_v, acc_sh, sem):
    c = lax.axis_index("c")
    s = lax.axis_index("s")
    pltpu.sync_copy(z2_hbm.at[pl.ds(s * ROWS, ROWS)],
                    acc_sh.at[pl.ds(s * ROWS, ROWS)])
    plsc.subcore_barrier()

    tbase = s * EPT_AGG

    def body(j, carry):
        e = tbase + j * CHUNK
        pltpu.sync_copy(src2_hbm.at[pl.ds(c * EP + e, CHUNK)], sidx_v)
        pltpu.sync_copy(dst_hbm.at[pl.ds(e, CHUNK)], didx_v)
        pltpu.async_copy(table_hbm.at[sidx_v], rows_v, sem).wait()
        pltpu.sync_copy(rows_v, acc_sh.at[didx_v], add=True)
        return carry

    lax.fori_loop(0, EPT_AGG // CHUNK, body, 0)
    plsc.subcore_barrier()
    pltpu.sync_copy(acc_sh.at[pl.ds(s * ROWS, ROWS)],
                    acc_hbm.at[pl.ds(c * NP + s * ROWS, ROWS)])


def _mm_body(x_ref, w_ref, deg_ref, o_ref):
    h = jnp.dot(x_ref[...], w_ref[...], preferred_element_type=jnp.float32)
    deg = deg_ref[0, :] + deg_ref[1, :] + 1.0
    dis = lax.rsqrt(deg)
    o_ref[...] = h * dis[:, None]


_mm_call = pl.pallas_call(
    _mm_body,
    grid=(RB, 2),
    in_specs=[
        pl.BlockSpec((BLK, D_IN), lambda i, c: (i, 0)),
        pl.BlockSpec((D_IN, 128), lambda i, c: (0, c)),
        pl.BlockSpec((2, BLK), lambda i, c: (0, i)),
    ],
    out_specs=pl.BlockSpec((BLK, 128), lambda i, c: (c * RB + i, 0)),
    out_shape=jax.ShapeDtypeStruct((2 * NP, 128), jnp.float32),
)


def _ep_body(acc_ref, g_ref, deg_ref, b_ref, o_ref):
    deg = deg_ref[0, :] + deg_ref[1, :] + 1.0
    dis = lax.rsqrt(deg)
    o_ref[...] = (acc_ref[...] + g_ref[...]) * dis[:, None] + b_ref[...]


_ep_call = pl.pallas_call(
    _ep_body,
    grid=(RB, 2),
    in_specs=[
        pl.BlockSpec((BLK, 128), lambda i, c: (c * RB + i, 0)),
        pl.BlockSpec((BLK, 128), lambda i, c: (c * RB + i, 0)),
        pl.BlockSpec((2, BLK), lambda i, c: (0, i)),
        pl.BlockSpec((1, 128), lambda i, c: (0, c)),
    ],
    out_specs=pl.BlockSpec((BLK, 128), lambda i, c: (i, c)),
    out_shape=jax.ShapeDtypeStruct((NP, D_OUT), jnp.float32),
)


def kernel(x, edge_index, W, b):
    ei = edge_index.astype(jnp.int32)
    src = ei[0]
    dst = ei[1]
    pad = EP - E
    srcp = jnp.concatenate([src, jnp.full((pad,), N, jnp.int32)])
    dstp = jnp.concatenate([dst, jnp.full((pad,), N, jnp.int32)])
    src2 = jnp.concatenate([srcp, srcp + NP])

    xp = jnp.zeros((NP, D_IN), jnp.float32).at[:N].set(x)
    z1 = jnp.zeros((NP,), jnp.float32)
    z2 = jnp.zeros((NP, 128), jnp.float32)
    ones = jnp.ones((CHUNK,), jnp.float32)

    deg2 = _deg_kernel(dstp, z1, ones).reshape(2, NP)
    table = _mm_call(xp, W, deg2)
    acc2 = _agg_kernel(table, src2, dstp, z2)
    outp = _ep_call(acc2, table, deg2, b.reshape(1, D_OUT))
    return outp[:N]


# speedup vs baseline: 9.2146x; 1.2834x over previous
import functools

import jax
import jax.numpy as jnp
from jax import lax
from jax.experimental import pallas as pl
from jax.experimental.pallas import tpu as pltpu
from jax.experimental.pallas import tpu_sc as plsc

N = 10000
D_IN = 256
D_OUT = 256
E = 160000

NP = 10240
ROWS = NP // 16
EP = 163840
CHUNK = 128
EPT_AGG = EP // 16
EPT_DEG = EP // 32
BLK = 512
RB = NP // BLK

_MESH = plsc.VectorSubcoreMesh(core_axis_name="c", subcore_axis_name="s")


DCPT = EPT_DEG // CHUNK


@functools.partial(
    pl.kernel,
    out_type=jax.ShapeDtypeStruct((2 * NP,), jnp.float32),
    mesh=_MESH,
    scratch_types=[
        pltpu.VMEM((DCPT, CHUNK), jnp.int32),
        pltpu.VMEM((CHUNK,), jnp.float32),
        pltpu.VMEM_SHARED((NP,), jnp.float32),
        pltpu.SemaphoreType.DMA,
    ],
)
def _deg_kernel(dst4_hbm, z1_hbm, ones_hbm, deg_hbm, idx_v, ones_v, acc_sh,
                isem):
    c = lax.axis_index("c")
    s = lax.axis_index("s")
    wid = c * 16 + s
    d1 = pltpu.async_copy(dst4_hbm.at[wid], idx_v, isem)
    pltpu.sync_copy(z1_hbm, acc_sh.at[pl.ds(s * ROWS, ROWS)])
    pltpu.sync_copy(ones_hbm, ones_v)
    d1.wait()
    plsc.subcore_barrier()

    def body(j, carry):
        pltpu.sync_copy(ones_v, acc_sh.at[idx_v.at[j]], add=True)
        return carry

    lax.fori_loop(0, DCPT, body, 0)
    plsc.subcore_barrier()
    pltpu.sync_copy(acc_sh.at[pl.ds(s * ROWS, ROWS)],
                    deg_hbm.at[pl.ds(c * NP + s * ROWS, ROWS)])


CPT = EPT_AGG // CHUNK
NBUF = 4


@functools.partial(
    pl.kernel,
    out_type=jax.ShapeDtypeStruct((2 * NP, 128), jnp.float32),
    mesh=_MESH,
    scratch_types=[
        pltpu.VMEM((CPT, CHUNK), jnp.int32),
        pltpu.VMEM((CPT, CHUNK), jnp.int32),
        pltpu.VMEM((CHUNK, 128), jnp.float32),
        pltpu.VMEM_SHARED((NP, 128), jnp.float32),
        pltpu.SemaphoreType.DMA,
    ],
)
def _agg_kernel(table_hbm, src3_hbm, dst3_hbm, z2_hbm, acc_hbm,
                sidx_v, didx_v, rows_v, acc_sh, sem):
    c = lax.axis_index("c")
    s = lax.axis_index("s")
    wid = c * 16 + s
    pltpu.sync_copy(src3_hbm.at[wid], sidx_v)
    pltpu.sync_copy(dst3_hbm.at[s], didx_v)
    pltpu.sync_copy(z2_hbm, acc_sh.at[pl.ds(s * ROWS, ROWS)])
    plsc.subcore_barrier()

    def body(j, carry):
        pltpu.async_copy(table_hbm.at[sidx_v.at[j]], rows_v, sem).wait()
        pltpu.sync_copy(rows_v, acc_sh.at[didx_v.at[j]], add=True)
        return carry

    lax.fori_loop(0, CPT, body, 0)
    plsc.subcore_barrier()
    pltpu.sync_copy(acc_sh.at[pl.ds(s * ROWS, ROWS)],
                    acc_hbm.at[pl.ds(c * NP + s * ROWS, ROWS)])


def _mm_body(x_ref, w_ref, deg_ref, o_ref):
    h = jnp.dot(x_ref[...], w_ref[...], preferred_element_type=jnp.float32)
    deg = deg_ref[0, :] + deg_ref[1, :] + 1.0
    dis = lax.rsqrt(deg)
    o_ref[...] = h * dis[:, None]


_mm_call = pl.pallas_call(
    _mm_body,
    grid=(RB, 2),
    in_specs=[
        pl.BlockSpec((BLK, D_IN), lambda i, c: (i, 0)),
        pl.BlockSpec((D_IN, 128), lambda i, c: (0, c)),
        pl.BlockSpec((2, BLK), lambda i, c: (0, i)),
    ],
    out_specs=pl.BlockSpec((BLK, 128), lambda i, c: (c * RB + i, 0)),
    out_shape=jax.ShapeDtypeStruct((2 * NP, 128), jnp.float32),
)


def _ep_body(acc_ref, g_ref, deg_ref, b_ref, o_ref):
    deg = deg_ref[0, :] + deg_ref[1, :] + 1.0
    dis = lax.rsqrt(deg)
    o_ref[...] = (acc_ref[...] + g_ref[...]) * dis[:, None] + b_ref[...]


_ep_call = pl.pallas_call(
    _ep_body,
    grid=(RB, 2),
    in_specs=[
        pl.BlockSpec((BLK, 128), lambda i, c: (c * RB + i, 0)),
        pl.BlockSpec((BLK, 128), lambda i, c: (c * RB + i, 0)),
        pl.BlockSpec((2, BLK), lambda i, c: (0, i)),
        pl.BlockSpec((1, 128), lambda i, c: (0, c)),
    ],
    out_specs=pl.BlockSpec((BLK, 128), lambda i, c: (i, c)),
    out_shape=jax.ShapeDtypeStruct((NP, D_OUT), jnp.float32),
)


def kernel(x, edge_index, W, b):
    ei = edge_index.astype(jnp.int32)
    src = ei[0]
    dst = ei[1]
    pad = EP - E
    srcp = jnp.concatenate([src, jnp.full((pad,), N, jnp.int32)])
    dstp = jnp.concatenate([dst, jnp.full((pad,), N, jnp.int32)])
    src3 = jnp.concatenate([srcp, srcp + NP]).reshape(32, CPT, CHUNK)
    dst3 = dstp.reshape(16, CPT, CHUNK)
    dst4 = dstp.reshape(32, DCPT, CHUNK)

    xp = jnp.concatenate([x, jnp.zeros((NP - N, D_IN), jnp.float32)])
    z1 = jnp.zeros((ROWS,), jnp.float32)
    z2 = jnp.zeros((ROWS, 128), jnp.float32)
    ones = jnp.ones((CHUNK,), jnp.float32)

    deg2 = _deg_kernel(dst4, z1, ones).reshape(2, NP)
    table = _mm_call(xp, W, deg2)
    acc2 = _agg_kernel(table, src3, dst3, z2)
    outp = _ep_call(acc2, table, deg2, b.reshape(1, D_OUT))
    return outp[:N]
